# Initial kernel scaffold; baseline (speedup 1.0000x reference)
import jax
import jax.numpy as jnp
from jax.experimental import pallas as pl


def kernel(z, edge_index, W, b):
    raise NotImplementedError("write your pallas kernel here")


# speedup vs baseline: 3.2128x; 3.2128x over previous
import functools

import jax
import jax.numpy as jnp
from jax.experimental import pallas as pl
from jax.experimental.pallas import tpu as pltpu

N_PAD = 10240
BM = 1024
BN = 1024
LATENT = 128


def _decode_body(h_ref, g_ref, o_ref):
    o_ref[...] = jax.lax.dot_general(
        h_ref[...], g_ref[...],
        dimension_numbers=(((1,), (1,)), ((), ())),
        preferred_element_type=jnp.float32,
    )


def _decode(h_pad):
    grid = (N_PAD // BM, N_PAD // BN)
    return pl.pallas_call(
        _decode_body,
        grid=grid,
        in_specs=[
            pl.BlockSpec((BM, LATENT), lambda i, j: (i, 0)),
            pl.BlockSpec((BN, LATENT), lambda i, j: (j, 0)),
        ],
        out_specs=pl.BlockSpec((BM, BN), lambda i, j: (i, j)),
        out_shape=jax.ShapeDtypeStruct((N_PAD, N_PAD), jnp.float32),
    )(h_pad, h_pad)


def kernel(z, edge_index, W, b):
    n = z.shape[0]
    src = edge_index[0].astype(jnp.int32)
    dst = edge_index[1].astype(jnp.int32)

    deg = jnp.ones((n,), jnp.float32).at[dst].add(1.0)
    dinv = jax.lax.rsqrt(deg)

    xs = dinv[:, None] * (z @ W)
    agg = xs.at[dst].add(xs[src])
    h = jax.nn.relu(dinv[:, None] * agg + b)

    h_pad = jnp.zeros((N_PAD, LATENT), jnp.float32).at[:n].set(h)
    adj = _decode(h_pad)
    return adj[:n, :n]


# speedup vs baseline: 3.5784x; 1.1138x over previous
import functools

import jax
import jax.numpy as jnp
from jax.experimental import pallas as pl
from jax.experimental.pallas import tpu as pltpu

BM = 1024
BN = 1024
LATENT = 128


def _decode_body(h_ref, g_ref, o_ref):
    o_ref[...] = jax.lax.dot_general(
        h_ref[...], g_ref[...],
        dimension_numbers=(((1,), (1,)), ((), ())),
        preferred_element_type=jnp.float32,
    )


def _decode(h):
    n = h.shape[0]
    grid = (pl.cdiv(n, BM), pl.cdiv(n, BN))
    return pl.pallas_call(
        _decode_body,
        grid=grid,
        in_specs=[
            pl.BlockSpec((BM, LATENT), lambda i, j: (i, 0)),
            pl.BlockSpec((BN, LATENT), lambda i, j: (j, 0)),
        ],
        out_specs=pl.BlockSpec((BM, BN), lambda i, j: (i, j)),
        out_shape=jax.ShapeDtypeStruct((n, n), jnp.float32),
    )(h, h)


def kernel(z, edge_index, W, b):
    n = z.shape[0]
    src = edge_index[0].astype(jnp.int32)
    dst = edge_index[1].astype(jnp.int32)

    deg = jnp.ones((n,), jnp.float32).at[dst].add(1.0)
    dinv = jax.lax.rsqrt(deg)

    xs = dinv[:, None] * (z @ W)
    agg = xs.at[dst].add(xs[src])
    h = jax.nn.relu(dinv[:, None] * agg + b)
    return _decode(h)


# speedup vs baseline: 8.9011x; 2.4875x over previous
import functools

import jax
import jax.numpy as jnp
from jax import lax
from jax.experimental import pallas as pl
from jax.experimental.pallas import tpu as pltpu
from jax.experimental.pallas import tpu_sc as plsc

N_NODES = 10000
N_EDGES = 320000
LATENT = 128

NUM_CORES = 2
NUM_SUBCORES = 16
NUM_WORKERS = NUM_CORES * NUM_SUBCORES
EDGES_PER_WORKER = N_EDGES // NUM_WORKERS
CHUNK = 80
CHUNKS_PER_WORKER = EDGES_PER_WORKER // CHUNK
SLAB = 632
SLAB_LAST = N_NODES - 15 * SLAB

BM = 1024
BN = 1024


def _sc_body(xs_hbm, src_hbm, dst_hbm, zrows_hbm, out_hbm,
             sidx, didx, rows, acc, sem):
    c = lax.axis_index("c")
    s = lax.axis_index("s")

    @pl.when(s < NUM_SUBCORES - 1)
    def _():
        pltpu.sync_copy(zrows_hbm, acc.at[pl.ds(s * SLAB, SLAB)])

    @pl.when(s == NUM_SUBCORES - 1)
    def _():
        pltpu.sync_copy(zrows_hbm.at[pl.ds(0, SLAB_LAST)],
                        acc.at[pl.ds(15 * SLAB, SLAB_LAST)])

    plsc.subcore_barrier()

    base0 = (c * NUM_SUBCORES + s) * EDGES_PER_WORKER

    def step(i, _):
        base = pl.multiple_of(base0 + i * CHUNK, 8)
        pltpu.sync_copy(src_hbm.at[pl.ds(base, CHUNK)], sidx)
        pltpu.sync_copy(dst_hbm.at[pl.ds(base, CHUNK)], didx)
        pltpu.async_copy(xs_hbm.at[sidx], rows, sem).wait()
        pltpu.sync_copy(rows, acc.at[didx], add=True)
        return _

    lax.fori_loop(0, CHUNKS_PER_WORKER, step, None)
    plsc.subcore_barrier()

    @pl.when(s < NUM_SUBCORES - 1)
    def _():
        r0 = s * SLAB
        pltpu.sync_copy(acc.at[pl.ds(r0, SLAB)],
                        out_hbm.at[pl.ds(c * N_NODES + r0, SLAB)])

    @pl.when(s == NUM_SUBCORES - 1)
    def _():
        r0 = 15 * SLAB
        pltpu.sync_copy(acc.at[pl.ds(r0, SLAB_LAST)],
                        out_hbm.at[pl.ds(c * N_NODES + r0, SLAB_LAST)])


@functools.partial(jax.jit, static_argnames=())
def _sc_scatter(xs, src, dst, zrows):
    mesh = plsc.VectorSubcoreMesh(core_axis_name="c", subcore_axis_name="s")
    k = pl.kernel(
        _sc_body,
        mesh=mesh,
        out_type=jax.ShapeDtypeStruct((NUM_CORES * N_NODES, LATENT),
                                      jnp.float32),
        scratch_types=[
            pltpu.VMEM((CHUNK,), jnp.int32),
            pltpu.VMEM((CHUNK,), jnp.int32),
            pltpu.VMEM((CHUNK, LATENT), jnp.float32),
            pltpu.VMEM_SHARED((N_NODES, LATENT), jnp.float32),
            pltpu.SemaphoreType.DMA,
        ],
    )
    return k(xs, src, dst, zrows)


def _decode_body(h_ref, g_ref, o_ref):
    o_ref[...] = jax.lax.dot_general(
        h_ref[...], g_ref[...],
        dimension_numbers=(((1,), (1,)), ((), ())),
        preferred_element_type=jnp.float32,
    )


def _decode(h):
    n = h.shape[0]
    grid = (pl.cdiv(n, BM), pl.cdiv(n, BN))
    return pl.pallas_call(
        _decode_body,
        grid=grid,
        in_specs=[
            pl.BlockSpec((BM, LATENT), lambda i, j: (i, 0)),
            pl.BlockSpec((BN, LATENT), lambda i, j: (j, 0)),
        ],
        out_specs=pl.BlockSpec((BM, BN), lambda i, j: (i, j)),
        out_shape=jax.ShapeDtypeStruct((n, n), jnp.float32),
    )(h, h)


def kernel(z, edge_index, W, b):
    n = z.shape[0]
    src = edge_index[0].astype(jnp.int32)
    dst = edge_index[1].astype(jnp.int32)

    deg = jnp.ones((n,), jnp.float32).at[dst].add(1.0)
    dinv = jax.lax.rsqrt(deg)

    xs = dinv[:, None] * (z @ W)
    zrows = jnp.zeros((SLAB, LATENT), jnp.float32)
    part = _sc_scatter(xs, src, dst, zrows)
    agg = part[:n] + part[n:] + xs
    h = jax.nn.relu(dinv[:, None] * agg + b)
    return _decode(h)


# speedup vs baseline: 9.2913x; 1.0438x over previous
import functools

import jax
import jax.numpy as jnp
from jax import lax
from jax.experimental import pallas as pl
from jax.experimental.pallas import tpu as pltpu
from jax.experimental.pallas import tpu_sc as plsc

N_NODES = 10000
N_EDGES = 320000
LATENT = 128

NUM_CORES = 2
NUM_SUBCORES = 16
NUM_WORKERS = NUM_CORES * NUM_SUBCORES
EDGES_PER_WORKER = N_EDGES // NUM_WORKERS
CHUNK = 80
CHUNKS_PER_WORKER = EDGES_PER_WORKER // CHUNK
SLAB = 632
SLAB_LAST = N_NODES - 15 * SLAB

BM = 2048
BN = 2048


def _sc_body(xs_hbm, src_hbm, dst_hbm, zrows_hbm, out_hbm,
             sidx, didx, rows, acc, sem):
    c = lax.axis_index("c")
    s = lax.axis_index("s")

    @pl.when(s < NUM_SUBCORES - 1)
    def _():
        pltpu.sync_copy(zrows_hbm, acc.at[pl.ds(s * SLAB, SLAB)])

    @pl.when(s == NUM_SUBCORES - 1)
    def _():
        pltpu.sync_copy(zrows_hbm.at[pl.ds(0, SLAB_LAST)],
                        acc.at[pl.ds(15 * SLAB, SLAB_LAST)])

    plsc.subcore_barrier()

    base0 = (c * NUM_SUBCORES + s) * EDGES_PER_WORKER

    def step(i, _):
        base = pl.multiple_of(base0 + i * CHUNK, 8)
        pltpu.sync_copy(src_hbm.at[pl.ds(base, CHUNK)], sidx)
        pltpu.sync_copy(dst_hbm.at[pl.ds(base, CHUNK)], didx)
        pltpu.async_copy(xs_hbm.at[sidx], rows, sem).wait()
        pltpu.sync_copy(rows, acc.at[didx], add=True)
        return _

    lax.fori_loop(0, CHUNKS_PER_WORKER, step, None)
    plsc.subcore_barrier()

    @pl.when(s < NUM_SUBCORES - 1)
    def _():
        r0 = s * SLAB
        pltpu.sync_copy(acc.at[pl.ds(r0, SLAB)],
                        out_hbm.at[pl.ds(c * N_NODES + r0, SLAB)])

    @pl.when(s == NUM_SUBCORES - 1)
    def _():
        r0 = 15 * SLAB
        pltpu.sync_copy(acc.at[pl.ds(r0, SLAB_LAST)],
                        out_hbm.at[pl.ds(c * N_NODES + r0, SLAB_LAST)])


@functools.partial(jax.jit, static_argnames=())
def _sc_scatter(xs, src, dst, zrows):
    mesh = plsc.VectorSubcoreMesh(core_axis_name="c", subcore_axis_name="s")
    k = pl.kernel(
        _sc_body,
        mesh=mesh,
        out_type=jax.ShapeDtypeStruct((NUM_CORES * N_NODES, LATENT),
                                      jnp.float32),
        scratch_types=[
            pltpu.VMEM((CHUNK,), jnp.int32),
            pltpu.VMEM((CHUNK,), jnp.int32),
            pltpu.VMEM((CHUNK, LATENT), jnp.float32),
            pltpu.VMEM_SHARED((N_NODES, LATENT), jnp.float32),
            pltpu.SemaphoreType.DMA,
        ],
    )
    return k(xs, src, dst, zrows)


def _decode_body(h_ref, g_ref, o_ref):
    o_ref[...] = jax.lax.dot_general(
        h_ref[...], g_ref[...],
        dimension_numbers=(((1,), (1,)), ((), ())),
        preferred_element_type=jnp.float32,
    )


def _decode(h):
    n = h.shape[0]
    grid = (pl.cdiv(n, BM), pl.cdiv(n, BN))
    return pl.pallas_call(
        _decode_body,
        grid=grid,
        in_specs=[
            pl.BlockSpec((BM, LATENT), lambda i, j: (i, 0)),
            pl.BlockSpec((BN, LATENT), lambda i, j: (j, 0)),
        ],
        out_specs=pl.BlockSpec((BM, BN), lambda i, j: (i, j)),
        out_shape=jax.ShapeDtypeStruct((n, n), jnp.float32),
    )(h, h)


def kernel(z, edge_index, W, b):
    n = z.shape[0]
    src = edge_index[0].astype(jnp.int32)
    dst = edge_index[1].astype(jnp.int32)

    deg = jnp.ones((n,), jnp.float32).at[dst].add(1.0)
    dinv = jax.lax.rsqrt(deg)

    xs = dinv[:, None] * (z @ W)
    zrows = jnp.zeros((SLAB, LATENT), jnp.float32)
    part = _sc_scatter(xs, src, dst, zrows)
    agg = part[:n] + part[n:] + xs
    h = jax.nn.relu(dinv[:, None] * agg + b)
    return _decode(h)


# speedup vs baseline: 9.3288x; 1.0040x over previous
import functools

import jax
import jax.numpy as jnp
from jax import lax
from jax.experimental import pallas as pl
from jax.experimental.pallas import tpu as pltpu
from jax.experimental.pallas import tpu_sc as plsc

N_NODES = 10000
N_EDGES = 320000
LATENT = 128

NUM_CORES = 2
NUM_SUBCORES = 16
NUM_WORKERS = NUM_CORES * NUM_SUBCORES
EDGES_PER_WORKER = N_EDGES // NUM_WORKERS
CHUNK = 80
CHUNKS_PER_WORKER = EDGES_PER_WORKER // CHUNK
SLAB = 632
SLAB_LAST = N_NODES - 15 * SLAB

BM = 2048
BN = 2048


def _sc_body(xs_hbm, src_hbm, dst_hbm, zrows_hbm, out_hbm,
             sidx, didx, rows, acc, sem):
    c = lax.axis_index("c")
    s = lax.axis_index("s")

    @pl.when(s < NUM_SUBCORES - 1)
    def _():
        pltpu.sync_copy(zrows_hbm, acc.at[pl.ds(s * SLAB, SLAB)])

    @pl.when(s == NUM_SUBCORES - 1)
    def _():
        pltpu.sync_copy(zrows_hbm.at[pl.ds(0, SLAB_LAST)],
                        acc.at[pl.ds(15 * SLAB, SLAB_LAST)])

    plsc.subcore_barrier()

    base0 = (c * NUM_SUBCORES + s) * EDGES_PER_WORKER

    def step(i, _):
        base = pl.multiple_of(base0 + i * CHUNK, 8)
        pltpu.sync_copy(src_hbm.at[pl.ds(base, CHUNK)], sidx)
        pltpu.sync_copy(dst_hbm.at[pl.ds(base, CHUNK)], didx)
        pltpu.async_copy(xs_hbm.at[sidx], rows, sem).wait()
        pltpu.sync_copy(rows, acc.at[didx], add=True)
        return _

    lax.fori_loop(0, CHUNKS_PER_WORKER, step, None)
    plsc.subcore_barrier()

    @pl.when(s < NUM_SUBCORES - 1)
    def _():
        r0 = s * SLAB
        pltpu.sync_copy(acc.at[pl.ds(r0, SLAB)],
                        out_hbm.at[pl.ds(c * N_NODES + r0, SLAB)])

    @pl.when(s == NUM_SUBCORES - 1)
    def _():
        r0 = 15 * SLAB
        pltpu.sync_copy(acc.at[pl.ds(r0, SLAB_LAST)],
                        out_hbm.at[pl.ds(c * N_NODES + r0, SLAB_LAST)])


@functools.partial(jax.jit, static_argnames=())
def _sc_scatter(xs, src, dst, zrows):
    mesh = plsc.VectorSubcoreMesh(core_axis_name="c", subcore_axis_name="s")
    k = pl.kernel(
        _sc_body,
        mesh=mesh,
        out_type=jax.ShapeDtypeStruct((NUM_CORES * N_NODES, LATENT),
                                      jnp.float32),
        scratch_types=[
            pltpu.VMEM((CHUNK,), jnp.int32),
            pltpu.VMEM((CHUNK,), jnp.int32),
            pltpu.VMEM((CHUNK, LATENT), jnp.float32),
            pltpu.VMEM_SHARED((N_NODES, LATENT), jnp.float32),
            pltpu.SemaphoreType.DMA,
        ],
    )
    return k(xs, src, dst, zrows)


def _decode_body(h_ref, g_ref, o_ref):
    o_ref[...] = jax.lax.dot_general(
        h_ref[...], g_ref[...],
        dimension_numbers=(((1,), (1,)), ((), ())),
        preferred_element_type=jnp.float32,
    )


def _decode(h):
    n = h.shape[0]
    assert h.dtype == jnp.bfloat16
    grid = (pl.cdiv(n, BM), pl.cdiv(n, BN))
    return pl.pallas_call(
        _decode_body,
        grid=grid,
        in_specs=[
            pl.BlockSpec((BM, LATENT), lambda i, j: (i, 0)),
            pl.BlockSpec((BN, LATENT), lambda i, j: (j, 0)),
        ],
        out_specs=pl.BlockSpec((BM, BN), lambda i, j: (i, j)),
        out_shape=jax.ShapeDtypeStruct((n, n), jnp.float32),
    )(h, h)


def kernel(z, edge_index, W, b):
    n = z.shape[0]
    src = edge_index[0].astype(jnp.int32)
    dst = edge_index[1].astype(jnp.int32)

    deg = jnp.ones((n,), jnp.float32).at[dst].add(1.0)
    dinv = jax.lax.rsqrt(deg)

    xs = dinv[:, None] * (z @ W)
    zrows = jnp.zeros((SLAB, LATENT), jnp.float32)
    part = _sc_scatter(xs, src, dst, zrows)
    agg = part[:n] + part[n:] + xs
    h = jax.nn.relu(dinv[:, None] * agg + b).astype(jnp.bfloat16)
    return _decode(h)
